# Initial kernel scaffold; baseline (speedup 1.0000x reference)
import jax
import jax.numpy as jnp
from jax.experimental import pallas as pl


def kernel(x, edge_index, W1, b1, W2, b2, Wfc, bfc):
    raise NotImplementedError("write your pallas kernel here")


# speedup vs baseline: 2.4783x; 2.4783x over previous
import functools

import jax
import jax.numpy as jnp
from jax import lax
from jax.experimental import pallas as pl
from jax.experimental.pallas import tpu as pltpu
from jax.experimental.pallas import tpu_sc as plsc

N = 10000
E = 160000
NP = 10240
RPT = NP // 16
NC, NS = 2, 16
K1 = 80
K2 = 40
DH = 128
D2 = 128

_HIGH = jax.lax.Precision.HIGHEST


@functools.cache
def _sc_mesh():
    return plsc.VectorSubcoreMesh(
        core_axis_name="c", subcore_axis_name="s",
        num_cores=NC, num_subcores=NS)


_HN = NP // 2
_HW = 16 * (_HN + 8)


def _deg_body(edge, out, idxall, hist):
    c = lax.axis_index("c")
    s = lax.axis_index("s")
    ept = E // NS
    iota = lax.iota(jnp.int32, 16)
    ones = jnp.ones((16,), jnp.float32)
    zero16 = jnp.zeros((16,), jnp.float32)
    pltpu.sync_copy(edge.at[pl.ds(c * E + s * ept, ept)], idxall)

    for p in range(2):
        def zhist(r, _):
            for j in range(8):
                hist[pl.ds(r * 128 + j * 16, 16)] = zero16
            return 0
        lax.fori_loop(0, _HW // 128, zhist, 0)

        lo = p * _HN

        def ebody(k, _):
            idx16 = idxall[pl.ds(k * 16, 16)]
            rel = idx16 - lo
            inr = (rel >= 0) & (rel < _HN)
            rel2 = jnp.where(inr, rel, _HN)
            plsc.addupdate_scatter(hist, [rel2 * 16 + iota], ones)
            return 0
        lax.fori_loop(0, ept // 16, ebody, 0)

        pltpu.sync_copy(
            hist, out.at[pl.ds((((c * NS) + s) * 2 + p) * _HW, _HW)])


@functools.cache
def _deg_call():
    return pl.kernel(
        _deg_body,
        out_type=jax.ShapeDtypeStruct((NC * NS * 2 * _HW,), jnp.float32),
        mesh=_sc_mesh(),
        compiler_params=pltpu.CompilerParams(needs_layout_passes=False),
        scratch_types=[
            pltpu.VMEM((E // NS,), jnp.int32),
            pltpu.VMEM((_HW,), jnp.float32),
        ],
    )


def _edge1_body(edge, table, out, srcv, dstv, rows, stage, acc, sem):
    c = lax.axis_index("c")
    s = lax.axis_index("s")
    zero16 = jnp.zeros((16,), jnp.float32)

    def init_zero(r, _):
        for j in range(DH // 16):
            stage[r, pl.ds(j * 16, 16)] = zero16
        return 0
    lax.fori_loop(0, 128, init_zero, 0)
    for t in range(RPT // 128):
        pltpu.sync_copy(stage, acc.at[pl.ds(s * RPT + t * 128, 128)])
    plsc.subcore_barrier()

    off = c * NP

    def chunk(g, _):
        base = s * (E // NS) + g * K1
        pltpu.sync_copy(edge.at[pl.ds(base, K1)], srcv)
        pltpu.sync_copy(edge.at[pl.ds(E + base, K1)], dstv)
        for j in range(K1 // 16):
            srcv[pl.ds(j * 16, 16)] = srcv[pl.ds(j * 16, 16)] + off
        pltpu.async_copy(table.at[srcv], rows, sem).wait()
        pltpu.sync_copy(rows, acc.at[dstv], add=True)
        return 0
    lax.fori_loop(0, E // NS // K1, chunk, 0)
    plsc.subcore_barrier()

    for t in range(RPT // 128):
        r0 = s * RPT + t * 128
        pltpu.sync_copy(acc.at[pl.ds(r0, 128)], stage)
        pltpu.sync_copy(stage, out.at[c, pl.ds(r0, 128)])


@functools.cache
def _edge1_call():
    return pl.kernel(
        _edge1_body,
        out_type=jax.ShapeDtypeStruct((NC, NP, DH), jnp.float32),
        mesh=_sc_mesh(),
        scratch_types=[
            pltpu.VMEM((K1,), jnp.int32),
            pltpu.VMEM((K1,), jnp.int32),
            pltpu.VMEM((K1, DH), jnp.float32),
            pltpu.VMEM((128, DH), jnp.float32),
            pltpu.VMEM_SHARED((NP, DH), jnp.float32),
            pltpu.SemaphoreType.DMA,
        ],
    )


def _edge2_body(edge, table, out, srcv, dstv, rows, stage, acc, sem):
    c = lax.axis_index("c")
    s = lax.axis_index("s")
    zero16 = jnp.zeros((16,), jnp.float32)
    ept = E // NC // NS

    def init_zero(r, _):
        for j in range(D2 // 16):
            stage[r, pl.ds(j * 16, 16)] = zero16
        return 0
    lax.fori_loop(0, 128, init_zero, 0)
    for t in range(RPT // 128):
        pltpu.sync_copy(stage, acc.at[pl.ds(s * RPT + t * 128, 128)])
    plsc.subcore_barrier()

    def chunk(g, _):
        base = c * (E // NC) + s * ept + g * K2
        pltpu.sync_copy(edge.at[pl.ds(base, K2)], srcv)
        pltpu.sync_copy(edge.at[pl.ds(E + base, K2)], dstv)
        pltpu.async_copy(table.at[srcv], rows, sem).wait()
        pltpu.sync_copy(rows, acc.at[dstv], add=True)
        return 0
    lax.fori_loop(0, ept // K2, chunk, 0)
    plsc.subcore_barrier()

    for t in range(RPT // 128):
        r0 = s * RPT + t * 128
        pltpu.sync_copy(acc.at[pl.ds(r0, 128)], stage)
        pltpu.sync_copy(stage, out.at[c, pl.ds(r0, 128)])


@functools.cache
def _edge2_call():
    return pl.kernel(
        _edge2_body,
        out_type=jax.ShapeDtypeStruct((NC, NP, D2), jnp.float32),
        mesh=_sc_mesh(),
        scratch_types=[
            pltpu.VMEM((K2,), jnp.int32),
            pltpu.VMEM((K2,), jnp.int32),
            pltpu.VMEM((K2, D2), jnp.float32),
            pltpu.VMEM((128, D2), jnp.float32),
            pltpu.VMEM_SHARED((NP, D2), jnp.float32),
            pltpu.SemaphoreType.DMA,
        ],
    )


def _deg_sum(deg_block):
    return jnp.sum(jnp.sum(deg_block, axis=-1), axis=0)


def _tc1_body(x_ref, w_ref, deg_ref, o_ref):
    nsrc = lax.rsqrt(jnp.maximum(_deg_sum(deg_ref[0]), 1.0))
    y = jnp.dot(x_ref[...], w_ref[...],
                preferred_element_type=jnp.float32, precision=_HIGH)
    o_ref[...] = y * nsrc[:, None]


_tc1_call = pl.pallas_call(
    _tc1_body,
    grid=(16, 2),
    in_specs=[
        pl.BlockSpec((RPT, 256), lambda i, j: (i, 0)),
        pl.BlockSpec((256, DH), lambda i, j: (0, j)),
        pl.BlockSpec((1, NS, RPT, 16), lambda i, j: (0, 0, i, 0)),
    ],
    out_specs=pl.BlockSpec((RPT, DH), lambda i, j: (j * 16 + i, 0)),
    out_shape=jax.ShapeDtypeStruct((NC * NP, DH), jnp.float32),
)


def _tc2_body(a1_ref, deg_ref, b1_ref, w2_ref, wfc_ref, o_ref):
    cat = jnp.concatenate([a1_ref[0], a1_ref[1]], axis=1)
    ndst = lax.rsqrt(jnp.maximum(_deg_sum(deg_ref[1]), 1.0))[:, None]
    nsrc = lax.rsqrt(jnp.maximum(_deg_sum(deg_ref[0]), 1.0))[:, None]
    h1 = jnp.maximum(cat * ndst + b1_ref[0][None, :], 0.0)
    w2fc = jnp.dot(w2_ref[...], wfc_ref[...],
                   preferred_element_type=jnp.float32, precision=_HIGH)
    y2 = jnp.dot(h1, w2fc, preferred_element_type=jnp.float32,
                 precision=_HIGH) * nsrc
    o_ref[...] = jnp.concatenate(
        [y2, jnp.zeros((y2.shape[0], D2 - 40), jnp.float32)], axis=1)


_tc2_call = pl.pallas_call(
    _tc2_body,
    grid=(16,),
    in_specs=[
        pl.BlockSpec((2, RPT, DH), lambda i: (0, i, 0)),
        pl.BlockSpec((2, NS, RPT, 16), lambda i: (0, 0, i, 0)),
        pl.BlockSpec((1, 256), lambda i: (0, 0)),
        pl.BlockSpec((256, DH), lambda i: (0, 0)),
        pl.BlockSpec((DH, 40), lambda i: (0, 0)),
    ],
    out_specs=pl.BlockSpec((RPT, D2), lambda i: (i, 0)),
    out_shape=jax.ShapeDtypeStruct((NP, D2), jnp.float32),
)


def _tc3_body(a2_ref, deg_ref, b2_ref, wfc_ref, bfc_ref, o_ref):
    ssum = a2_ref[0, :, 0:40] + a2_ref[1, :, 0:40]
    ndst = lax.rsqrt(jnp.maximum(_deg_sum(deg_ref[0]), 1.0))[:, None]
    bias = jnp.dot(b2_ref[...], wfc_ref[...],
                   preferred_element_type=jnp.float32, precision=_HIGH) \
        + bfc_ref[...]
    z = ssum * ndst + bias
    m = jnp.max(z, axis=1, keepdims=True)
    lse = jnp.log(jnp.sum(jnp.exp(z - m), axis=1, keepdims=True)) + m
    o_ref[...] = z - lse


_tc3_call = pl.pallas_call(
    _tc3_body,
    grid=(10,),
    in_specs=[
        pl.BlockSpec((2, 1000, D2), lambda i: (0, i, 0)),
        pl.BlockSpec((1, NS, 1000, 16), lambda i: (1, 0, i, 0)),
        pl.BlockSpec((1, DH), lambda i: (0, 0)),
        pl.BlockSpec((DH, 40), lambda i: (0, 0)),
        pl.BlockSpec((1, 40), lambda i: (0, 0)),
    ],
    out_specs=pl.BlockSpec((1000, 40), lambda i: (i, 0)),
    out_shape=jax.ShapeDtypeStruct((N, 40), jnp.float32),
)


def kernel(x, edge_index, W1, b1, W2, b2, Wfc, bfc):
    edge = edge_index.astype(jnp.int32).reshape(2 * E)
    deg4 = _deg_call()(edge).reshape(NC, NS, 2, _HN + 8, 16)
    deg = jnp.concatenate(
        [deg4[:, :, 0, :_HN, :], deg4[:, :, 1, :_HN, :]], axis=2)
    y1t = _tc1_call(x, W1, deg)
    a1 = _edge1_call()(edge, y1t)
    y2t = _tc2_call(a1, deg, b1[None, :], W2, Wfc)
    a2 = _edge2_call()(edge, y2t)
    return _tc3_call(a2, deg, b2[None, :], Wfc, bfc[None, :])


# speedup vs baseline: 3.3286x; 1.3431x over previous
import functools

import jax
import jax.numpy as jnp
from jax import lax
from jax.experimental import pallas as pl
from jax.experimental.pallas import tpu as pltpu
from jax.experimental.pallas import tpu_sc as plsc

N = 10000
E = 160000
NP = 10240
RPT = NP // 16
NC, NS = 2, 16
K1 = 40
K2 = 40
DH = 128
D2 = 128

_HIGH = jax.lax.Precision.HIGHEST


@functools.cache
def _sc_mesh():
    return plsc.VectorSubcoreMesh(
        core_axis_name="c", subcore_axis_name="s",
        num_cores=NC, num_subcores=NS)


_HN = NP // 2
_HW = 16 * (_HN + 8)


def _deg_body(edge, out, idxall, hist):
    c = lax.axis_index("c")
    s = lax.axis_index("s")
    ept = E // NS
    iota = lax.iota(jnp.int32, 16)
    ones = jnp.ones((16,), jnp.float32)
    zero16 = jnp.zeros((16,), jnp.float32)
    pltpu.sync_copy(edge.at[pl.ds(c * 2 * E + s * ept, ept)], idxall)

    for p in range(2):
        def zhist(r, _):
            for j in range(8):
                hist[pl.ds(r * 128 + j * 16, 16)] = zero16
            return 0
        lax.fori_loop(0, _HW // 128, zhist, 0)

        lo = p * _HN

        def ebody(k, _):
            idx16 = idxall[pl.ds(k * 16, 16)]
            rel = idx16 - lo
            inr = (rel >= 0) & (rel < _HN)
            rel2 = jnp.where(inr, rel, _HN)
            plsc.addupdate_scatter(hist, [rel2 * 16 + iota], ones)
            return 0
        lax.fori_loop(0, ept // 16, ebody, 0)

        pltpu.sync_copy(
            hist, out.at[pl.ds((((c * NS) + s) * 2 + p) * _HW, _HW)])


@functools.cache
def _deg_call():
    return pl.kernel(
        _deg_body,
        out_type=jax.ShapeDtypeStruct((NC * NS * 2 * _HW,), jnp.float32),
        mesh=_sc_mesh(),
        compiler_params=pltpu.CompilerParams(needs_layout_passes=False),
        scratch_types=[
            pltpu.VMEM((E // NS,), jnp.int32),
            pltpu.VMEM((_HW,), jnp.float32),
        ],
    )


NBUF = 5


def _make_edge(K, ept, width, use_core_offset):
    nchunks = ept // K
    assert nchunks % NBUF == 0

    def body(edge3, table, out, rows, sidx, didx, acc, gsem, isem, dsem):
        c = lax.axis_index("c")
        s = lax.axis_index("s")
        zero16 = jnp.zeros((16,), jnp.float32)
        if use_core_offset:
            sbase = c * E + s * ept
            dbase = 2 * E + s * ept
        else:
            half = c * (E // NC)
            sbase = half + s * ept
            dbase = 2 * E + half + s * ept

        def init_zero(r, _):
            for b in range(NBUF):
                for j in range(width // 16):
                    rows[b][r, pl.ds(j * 16, 16)] = zero16
            return 0
        lax.fori_loop(0, K, init_zero, 0)
        for t in range(RPT // K):
            pltpu.sync_copy(rows[t % NBUF],
                            acc.at[pl.ds(s * RPT + t * K, K)])
        plsc.subcore_barrier()

        def group(t, _):
            start0 = t * (NBUF * K)
            idescs = []
            ddescs = []
            for b in range(NBUF):
                idescs.append(pltpu.async_copy(
                    edge3.at[pl.ds(sbase + start0 + b * K, K)],
                    sidx[b], isem[b]))
                ddescs.append(pltpu.async_copy(
                    edge3.at[pl.ds(dbase + start0 + b * K, K)],
                    didx[b], dsem[b]))
            gathers = []
            for b in range(NBUF):
                idescs[b].wait()
                gathers.append(
                    pltpu.async_copy(table.at[sidx[b]], rows[b], gsem[b]))
            for b in range(NBUF):
                gathers[b].wait()
                ddescs[b].wait()
                pltpu.sync_copy(rows[b], acc.at[didx[b]], add=True)
            return 0
        lax.fori_loop(0, nchunks // NBUF, group, 0)
        plsc.subcore_barrier()

        for t in range(RPT // K):
            r0 = s * RPT + t * K
            b = t % NBUF
            pltpu.sync_copy(acc.at[pl.ds(r0, K)], rows[b])
            pltpu.sync_copy(rows[b], out.at[c, pl.ds(r0, K)])

    return pl.kernel(
        body,
        out_type=jax.ShapeDtypeStruct((NC, NP, width), jnp.float32),
        mesh=_sc_mesh(),
        scratch_types=[
            [pltpu.VMEM((K, width), jnp.float32) for _ in range(NBUF)],
            [pltpu.VMEM((K,), jnp.int32) for _ in range(NBUF)],
            [pltpu.VMEM((K,), jnp.int32) for _ in range(NBUF)],
            pltpu.VMEM_SHARED((NP, width), jnp.float32),
            [pltpu.SemaphoreType.DMA for _ in range(NBUF)],
            [pltpu.SemaphoreType.DMA for _ in range(NBUF)],
            [pltpu.SemaphoreType.DMA for _ in range(NBUF)],
        ],
    )


@functools.cache
def _edge1_call():
    return _make_edge(K1, E // NS, DH, use_core_offset=True)


@functools.cache
def _edge2_call():
    return _make_edge(K2, E // NC // NS, D2, use_core_offset=False)


def _deg_sum(deg_block):
    return jnp.sum(jnp.sum(deg_block, axis=-1), axis=0)


def _tc1_body(x_ref, w_ref, deg_ref, o_ref):
    nsrc = lax.rsqrt(jnp.maximum(_deg_sum(deg_ref[0]), 1.0))
    y = jnp.dot(x_ref[...], w_ref[...],
                preferred_element_type=jnp.float32, precision=_HIGH)
    o_ref[...] = y * nsrc[:, None]


_tc1_call = pl.pallas_call(
    _tc1_body,
    grid=(16, 2),
    in_specs=[
        pl.BlockSpec((RPT, 256), lambda i, j: (i, 0)),
        pl.BlockSpec((256, DH), lambda i, j: (0, j)),
        pl.BlockSpec((1, NS, RPT, 16), lambda i, j: (0, 0, i, 0)),
    ],
    out_specs=pl.BlockSpec((RPT, DH), lambda i, j: (j * 16 + i, 0)),
    out_shape=jax.ShapeDtypeStruct((NC * NP, DH), jnp.float32),
)


def _tc2_body(a1_ref, deg_ref, b1_ref, w2_ref, wfc_ref, o_ref):
    cat = jnp.concatenate([a1_ref[0], a1_ref[1]], axis=1)
    ndst = lax.rsqrt(jnp.maximum(_deg_sum(deg_ref[1]), 1.0))[:, None]
    nsrc = lax.rsqrt(jnp.maximum(_deg_sum(deg_ref[0]), 1.0))[:, None]
    h1 = jnp.maximum(cat * ndst + b1_ref[0][None, :], 0.0)
    w2fc = jnp.dot(w2_ref[...], wfc_ref[...],
                   preferred_element_type=jnp.float32, precision=_HIGH)
    y2 = jnp.dot(h1, w2fc, preferred_element_type=jnp.float32,
                 precision=_HIGH) * nsrc
    o_ref[...] = jnp.concatenate(
        [y2, jnp.zeros((y2.shape[0], D2 - 40), jnp.float32)], axis=1)


_tc2_call = pl.pallas_call(
    _tc2_body,
    grid=(16,),
    in_specs=[
        pl.BlockSpec((2, RPT, DH), lambda i: (0, i, 0)),
        pl.BlockSpec((2, NS, RPT, 16), lambda i: (0, 0, i, 0)),
        pl.BlockSpec((1, 256), lambda i: (0, 0)),
        pl.BlockSpec((256, DH), lambda i: (0, 0)),
        pl.BlockSpec((DH, 40), lambda i: (0, 0)),
    ],
    out_specs=pl.BlockSpec((RPT, D2), lambda i: (i, 0)),
    out_shape=jax.ShapeDtypeStruct((NP, D2), jnp.float32),
)


def _tc3_body(a2_ref, deg_ref, b2_ref, wfc_ref, bfc_ref, o_ref):
    ssum = a2_ref[0, :, 0:40] + a2_ref[1, :, 0:40]
    ndst = lax.rsqrt(jnp.maximum(_deg_sum(deg_ref[0]), 1.0))[:, None]
    bias = jnp.dot(b2_ref[...], wfc_ref[...],
                   preferred_element_type=jnp.float32, precision=_HIGH) \
        + bfc_ref[...]
    z = ssum * ndst + bias
    m = jnp.max(z, axis=1, keepdims=True)
    lse = jnp.log(jnp.sum(jnp.exp(z - m), axis=1, keepdims=True)) + m
    o_ref[...] = z - lse


_tc3_call = pl.pallas_call(
    _tc3_body,
    grid=(10,),
    in_specs=[
        pl.BlockSpec((2, 1000, D2), lambda i: (0, i, 0)),
        pl.BlockSpec((1, NS, 1000, 16), lambda i: (1, 0, i, 0)),
        pl.BlockSpec((1, DH), lambda i: (0, 0)),
        pl.BlockSpec((DH, 40), lambda i: (0, 0)),
        pl.BlockSpec((1, 40), lambda i: (0, 0)),
    ],
    out_specs=pl.BlockSpec((1000, 40), lambda i: (i, 0)),
    out_shape=jax.ShapeDtypeStruct((N, 40), jnp.float32),
)


def kernel(x, edge_index, W1, b1, W2, b2, Wfc, bfc):
    src, dst = edge_index[0].astype(jnp.int32), edge_index[1].astype(jnp.int32)
    edge = jnp.concatenate([src, src + NP, dst])
    deg4 = _deg_call()(edge).reshape(NC, NS, 2, _HN + 8, 16)
    deg = jnp.concatenate(
        [deg4[:, :, 0, :_HN, :], deg4[:, :, 1, :_HN, :]], axis=2)
    y1t = _tc1_call(x, W1, deg)
    a1 = _edge1_call()(edge, y1t)
    y2t = _tc2_call(a1, deg, b1[None, :], W2, Wfc)
    a2 = _edge2_call()(edge, y2t)
    return _tc3_call(a2, deg, b2[None, :], Wfc, bfc[None, :])
